# Initial kernel scaffold; baseline (speedup 1.0000x reference)
import jax
import jax.numpy as jnp
from jax.experimental import pallas as pl


def kernel(node_features, edges, W1, b1, W2, b2, W_ih, W_hh, b_ih, b_hh):
    raise NotImplementedError("write your pallas kernel here")


# speedup vs baseline: 2.2423x; 2.2423x over previous
import functools

import jax
import jax.numpy as jnp
from jax import lax
from jax.experimental import pallas as pl
from jax.experimental.pallas import tpu as pltpu
from jax.experimental.pallas import tpu_sc as plsc

N = 10000
E = 320000
D = 128

NC = 2
NS = 16
L = 16

EPT = E // (NC * NS)
KB = 80
NBB = EPT // KB

NPAD = 10240
HALF = NPAD // 2
GARB = HALF
SROWS = 5248
DNS = SROWS // NS
WBS = HALF // NS
BLK = 1024


def _pre_body(x_ref, w1at_ref, w1bt_ref, b1_ref, a_ref, b_ref):
    x = x_ref[...]
    a_ref[...] = jnp.dot(x, w1at_ref[...], preferred_element_type=jnp.float32) + b1_ref[...]
    b_ref[...] = jnp.dot(x, w1bt_ref[...], preferred_element_type=jnp.float32)


def _precompute(x_pad, w1at, w1bt, b1_row):
    grid = (NPAD // BLK,)
    return pl.pallas_call(
        _pre_body,
        grid=grid,
        in_specs=[
            pl.BlockSpec((BLK, D), lambda i: (i, 0)),
            pl.BlockSpec((D, D), lambda i: (0, 0)),
            pl.BlockSpec((D, D), lambda i: (0, 0)),
            pl.BlockSpec((1, D), lambda i: (0, 0)),
        ],
        out_specs=[
            pl.BlockSpec((BLK, D), lambda i: (i, 0)),
            pl.BlockSpec((BLK, D), lambda i: (i, 0)),
        ],
        out_shape=[
            jax.ShapeDtypeStruct((NPAD, D), jnp.float32),
            jax.ShapeDtypeStruct((NPAD, D), jnp.float32),
        ],
    )(x_pad, w1at, w1bt, b1_row)


def _sc_body(a_hbm, b_hbm, src_hbm, tgt_hbm, zn_hbm, ones_hbm,
             hp_hbm, deg_hbm,
             srcv, tgtv, idxb, ra, rb, onesb, acc):
    cid = lax.axis_index("c")
    sid = lax.axis_index("s")

    pltpu.sync_copy(src_hbm.at[cid, sid], srcv)
    pltpu.sync_copy(tgt_hbm.at[cid, sid], tgtv)
    pltpu.sync_copy(ones_hbm, onesb)

    garbv = jnp.full((L,), GARB, jnp.int32)
    halfv = jnp.full((L,), HALF, jnp.int32)

    def remap(j, lower):
        for g in range(KB // L):
            s = pl.ds(g * L, L)
            t = tgtv[j, s]
            if lower:
                iv = jnp.minimum(t, garbv)
            else:
                iv = jnp.where(t >= halfv, t - halfv, garbv)
            idxb[0, s] = iv

    def sweep(lower, hidden, out_hbm):
        pltpu.sync_copy(zn_hbm, acc.at[pl.ds(sid * DNS, DNS)])
        plsc.subcore_barrier()

        def chunk(j, c):
            remap(j, lower)
            if hidden:
                pltpu.sync_copy(a_hbm.at[srcv.at[j]], ra)
                pltpu.sync_copy(b_hbm.at[tgtv.at[j]], rb)

                def row(r, c2):
                    for g in range(D // L):
                        s = pl.ds(g * L, L)
                        ra[r, s] = jnp.maximum(ra[r, s] + rb[r, s], 0.0)
                    return c2

                lax.fori_loop(0, KB, row, 0)
                pltpu.sync_copy(ra, acc.at[idxb.at[0]], add=True)
            else:
                pltpu.sync_copy(onesb, acc.at[idxb.at[0]], add=True)
            return c

        lax.fori_loop(0, NBB, chunk, 0)
        plsc.subcore_barrier()

        off = 0 if lower else HALF
        pltpu.sync_copy(acc.at[pl.ds(sid * WBS, WBS)],
                        out_hbm.at[cid, pl.ds(off + sid * WBS, WBS)])
        plsc.subcore_barrier()

    sweep(True, True, hp_hbm)
    sweep(False, True, hp_hbm)
    sweep(True, False, deg_hbm)
    sweep(False, False, deg_hbm)


def _sc_edge_stage(a_pad, b_pad, src4, tgt4, zn, ones):
    mesh = plsc.VectorSubcoreMesh(core_axis_name="c", subcore_axis_name="s")
    run = functools.partial(
        pl.kernel,
        mesh=mesh,
        out_type=[
            jax.ShapeDtypeStruct((NC, NPAD, D), jnp.float32),
            jax.ShapeDtypeStruct((NC, NPAD, D), jnp.float32),
        ],
        scratch_types=[
            pltpu.VMEM((NBB, KB), jnp.int32),
            pltpu.VMEM((NBB, KB), jnp.int32),
            pltpu.VMEM((1, KB), jnp.int32),
            pltpu.VMEM((KB, D), jnp.float32),
            pltpu.VMEM((KB, D), jnp.float32),
            pltpu.VMEM((KB, D), jnp.float32),
            pltpu.VMEM_SHARED((SROWS, D), jnp.float32),
        ],
    )(_sc_body)
    return run(a_pad, b_pad, src4, tgt4, zn, ones)


def _post_body(hp_ref, degs_ref, x_ref, w2t_ref, b2_ref,
               wiht_ref, whht_ref, bih_ref, bhh_ref, out_ref):
    h = hp_ref[0] + hp_ref[1]
    deg = degs_ref[0, :, 0] + degs_ref[1, :, 0]
    agg = (jnp.dot(h, w2t_ref[...], preferred_element_type=jnp.float32)
           + deg[:, None] * b2_ref[...])
    gi = jnp.dot(agg, wiht_ref[...], preferred_element_type=jnp.float32) + bih_ref[...]
    x = x_ref[...]
    gh = jnp.dot(x, whht_ref[...], preferred_element_type=jnp.float32) + bhh_ref[...]
    r = jax.nn.sigmoid(gi[:, :D] + gh[:, :D])
    z = jax.nn.sigmoid(gi[:, D:2 * D] + gh[:, D:2 * D])
    n = jnp.tanh(gi[:, 2 * D:] + r * gh[:, 2 * D:])
    out_ref[...] = (1.0 - z) * n + z * x


def _postprocess(hp, degs, x_pad, w2t, b2_row, wiht, whht, bih_row, bhh_row):
    grid = (NPAD // BLK,)
    return pl.pallas_call(
        _post_body,
        grid=grid,
        in_specs=[
            pl.BlockSpec((NC, BLK, D), lambda i: (0, i, 0)),
            pl.BlockSpec((NC, BLK, D), lambda i: (0, i, 0)),
            pl.BlockSpec((BLK, D), lambda i: (i, 0)),
            pl.BlockSpec((D, D), lambda i: (0, 0)),
            pl.BlockSpec((1, D), lambda i: (0, 0)),
            pl.BlockSpec((D, 3 * D), lambda i: (0, 0)),
            pl.BlockSpec((D, 3 * D), lambda i: (0, 0)),
            pl.BlockSpec((1, 3 * D), lambda i: (0, 0)),
            pl.BlockSpec((1, 3 * D), lambda i: (0, 0)),
        ],
        out_specs=pl.BlockSpec((BLK, D), lambda i: (i, 0)),
        out_shape=jax.ShapeDtypeStruct((NPAD, D), jnp.float32),
    )(hp, degs, x_pad, w2t, b2_row, wiht, whht, bih_row, bhh_row)


def kernel(node_features, edges, W1, b1, W2, b2, W_ih, W_hh, b_ih, b_hh):
    x_pad = jnp.pad(node_features, ((0, NPAD - N), (0, 0)))

    src4 = edges[:, 0].reshape(NC, NS, NBB, KB)
    tgt4 = edges[:, 2].reshape(NC, NS, NBB, KB)

    w1at = W1[:, :D].T
    w1bt = W1[:, D:].T
    a, b = _precompute(x_pad, w1at, w1bt, b1[None, :])

    zn = jnp.zeros((DNS, D), jnp.float32)
    ones = jnp.ones((KB, D), jnp.float32)

    hp, degout = _sc_edge_stage(a, b, src4, tgt4, zn, ones)

    out = _postprocess(hp, degout, x_pad, W2.T, b2[None, :],
                       W_ih.T, W_hh.T, b_ih[None, :], b_hh[None, :])
    return out[:N]
